# Initial kernel scaffold; baseline (speedup 1.0000x reference)
import jax
import jax.numpy as jnp
from jax.experimental import pallas as pl


def kernel(x, edge_index, lin_in_W, lin_in_b, conv_W0, conv_b0, conv_W1, conv_b1, conv_W2, conv_b2, lin_W0, lin_b0, lin_W1, lin_b1, lin_W2, lin_b2, ln_g0, ln_b0, ln_g1, ln_b1, ln_g2, ln_b2, bn_g0, bn_b0, bn_g1, bn_b1, bn_g2, bn_b2, pred_W, pred_b):
    raise NotImplementedError("write your pallas kernel here")


# speedup vs baseline: 7.7806x; 7.7806x over previous
import functools

import jax
import jax.numpy as jnp
from jax import lax
from jax.experimental import pallas as pl
from jax.experimental.pallas import tpu as pltpu
from jax.experimental.pallas import tpu_sc as plsc

N = 10000
E = 320000
HID = 128
OUT = 7

NW = 32
CHUNK = 128
EPT = 10240
NCH = EPT // CHUNK
NACC = 10112
RPT = NACC // 16


def _sc_degree_body(dst_hbm, zeros_hbm, ones_hbm, out_hbm, dst_v, ones_v, slab):
    cid = lax.axis_index("c")
    sid = lax.axis_index("s")
    wid = sid * 2 + cid
    pltpu.sync_copy(dst_hbm.at[wid], dst_v)
    pltpu.sync_copy(ones_hbm, ones_v)
    pltpu.sync_copy(zeros_hbm.at[pl.ds(sid * RPT, RPT)],
                    slab.at[pl.ds(sid * RPT, RPT)])
    plsc.subcore_barrier()

    def body(j, _):
        pltpu.sync_copy(ones_v, slab.at[dst_v.at[j]], add=True)
        return ()

    lax.fori_loop(0, NCH, body, ())
    plsc.subcore_barrier()
    pltpu.sync_copy(slab.at[pl.ds(sid * RPT, RPT)],
                    out_hbm.at[cid, pl.ds(sid * RPT, RPT)])


def _sc_agg_body(y_hbm, src_hbm, dst_hbm, zeros_hbm, out_hbm,
                 src_v, dst_v, buf, acc):
    cid = lax.axis_index("c")
    sid = lax.axis_index("s")
    wid = sid * 2 + cid
    pltpu.sync_copy(src_hbm.at[wid], src_v)
    pltpu.sync_copy(dst_hbm.at[wid], dst_v)
    pltpu.sync_copy(zeros_hbm.at[pl.ds(sid * RPT, RPT)],
                    acc.at[pl.ds(sid * RPT, RPT)])
    plsc.subcore_barrier()

    def body(j, _):
        pltpu.sync_copy(y_hbm.at[src_v.at[j]], buf)
        pltpu.sync_copy(buf, acc.at[dst_v.at[j]], add=True)
        return ()

    lax.fori_loop(0, NCH, body, ())
    plsc.subcore_barrier()
    pltpu.sync_copy(acc.at[pl.ds(sid * RPT, RPT)],
                    out_hbm.at[cid, pl.ds(sid * RPT, RPT)])


@functools.cache
def _sc_kernels():
    mesh = plsc.VectorSubcoreMesh(core_axis_name="c", subcore_axis_name="s")
    sc_degree = pl.kernel(
        _sc_degree_body,
        out_type=jax.ShapeDtypeStruct((2, NACC, HID), jnp.float32),
        mesh=mesh,
        scratch_types=[
            pltpu.VMEM((NCH, CHUNK), jnp.int32),
            pltpu.VMEM((CHUNK, HID), jnp.float32),
            pltpu.VMEM_SHARED((NACC, HID), jnp.float32),
        ],
    )
    sc_agg = pl.kernel(
        _sc_agg_body,
        out_type=jax.ShapeDtypeStruct((2, NACC, HID), jnp.float32),
        mesh=mesh,
        scratch_types=[
            pltpu.VMEM((NCH, CHUNK), jnp.int32),
            pltpu.VMEM((NCH, CHUNK), jnp.int32),
            pltpu.VMEM((CHUNK, HID), jnp.float32),
            pltpu.VMEM_SHARED((NACC, HID), jnp.float32),
        ],
    )
    return sc_degree, sc_agg


def _tc0_body(x_ref, slab_ref, liW_ref, lib_ref, cW_ref, lW_ref, lb_ref,
              h0_ref, y_ref, s_ref, dinv_ref):
    x = x_ref[...]
    deg = 1.0 + slab_ref[0, :N, 0:1] + slab_ref[1, :N, 0:1]
    dinv = lax.rsqrt(deg)
    dinv_ref[...] = dinv
    h0_ref[...] = jnp.maximum(
        jnp.dot(x, liW_ref[...], preferred_element_type=jnp.float32)
        + lib_ref[...], 0.0)
    y_ref[...] = dinv * jnp.dot(x, cW_ref[...],
                                preferred_element_type=jnp.float32)
    s_ref[...] = jnp.dot(x, lW_ref[...],
                         preferred_element_type=jnp.float32) + lb_ref[...]


def _norm_relu(agg_ref, y_ref, s_ref, dinv_ref, cb_ref,
               bng_ref, bnb_ref, lng_ref, lnb_ref):
    dinv = dinv_ref[...]
    agg = agg_ref[0, :N, :] + agg_ref[1, :N, :]
    c = dinv * (agg + y_ref[...]) + cb_ref[...]
    t = c + s_ref[...]
    m = jnp.mean(t, axis=0, keepdims=True)
    d = t - m
    v = jnp.mean(d * d, axis=0, keepdims=True)
    tb = bng_ref[...] * d * lax.rsqrt(v + 1e-5) + bnb_ref[...]
    m2 = jnp.mean(tb, axis=1, keepdims=True)
    d2 = tb - m2
    v2 = jnp.mean(d2 * d2, axis=1, keepdims=True)
    tl = lng_ref[...] * d2 * lax.rsqrt(v2 + 1e-5) + lnb_ref[...]
    return jnp.maximum(tl, 0.0)


def _tc_mid_body(agg_ref, y_ref, s_ref, dinv_ref, cb_ref,
                 bng_ref, bnb_ref, lng_ref, lnb_ref,
                 cW2_ref, lW2_ref, lb2_ref,
                 y2_ref, s2_ref):
    h = _norm_relu(agg_ref, y_ref, s_ref, dinv_ref, cb_ref,
                   bng_ref, bnb_ref, lng_ref, lnb_ref)
    dinv = dinv_ref[...]
    y2_ref[...] = dinv * jnp.dot(h, cW2_ref[...],
                                 preferred_element_type=jnp.float32)
    s2_ref[...] = jnp.dot(h, lW2_ref[...],
                          preferred_element_type=jnp.float32) + lb2_ref[...]


def _tc_final_body(agg_ref, y_ref, s_ref, dinv_ref, cb_ref,
                   bng_ref, bnb_ref, lng_ref, lnb_ref,
                   h0_ref, pW_ref, pb_ref, out_ref):
    h = _norm_relu(agg_ref, y_ref, s_ref, dinv_ref, cb_ref,
                   bng_ref, bnb_ref, lng_ref, lnb_ref)
    out_ref[...] = jnp.dot(h + h0_ref[...], pW_ref[...],
                           preferred_element_type=jnp.float32) + pb_ref[...]


_f32 = lambda *s: jax.ShapeDtypeStruct(s, jnp.float32)

_tc0 = pl.pallas_call(
    _tc0_body,
    out_shape=(_f32(N, HID), _f32(N, HID), _f32(N, HID), _f32(N, 1)),
)

_tc_mid = pl.pallas_call(
    _tc_mid_body,
    out_shape=(_f32(N, HID), _f32(N, HID)),
)

_tc_final = pl.pallas_call(
    _tc_final_body,
    out_shape=_f32(N, 8),
)


def kernel(x, edge_index,
           lin_in_W, lin_in_b,
           conv_W0, conv_b0, conv_W1, conv_b1, conv_W2, conv_b2,
           lin_W0, lin_b0, lin_W1, lin_b1, lin_W2, lin_b2,
           ln_g0, ln_b0, ln_g1, ln_b1, ln_g2, ln_b2,
           bn_g0, bn_b0, bn_g1, bn_b1, bn_g2, bn_b2,
           pred_W, pred_b):
    pad = NW * EPT - E
    src = jnp.concatenate(
        [edge_index[0], jnp.zeros((pad,), jnp.int32)]).reshape(NW, NCH, CHUNK)
    dst = jnp.concatenate(
        [edge_index[1], jnp.full((pad,), N, jnp.int32)]).reshape(NW, NCH, CHUNK)
    zeros128 = jnp.zeros((NACC, HID), jnp.float32)
    ones128 = jnp.ones((CHUNK, HID), jnp.float32)
    pW = jnp.zeros((HID, 8), jnp.float32).at[:, :OUT].set(pred_W)
    pb = jnp.zeros((8,), jnp.float32).at[:OUT].set(pred_b)

    _sc_degree, _sc_agg = _sc_kernels()
    slab = _sc_degree(dst, zeros128, ones128)
    h0, y, s, dinv = _tc0(x, slab, lin_in_W, lin_in_b, conv_W0, lin_W0, lin_b0)

    agg = _sc_agg(y, src, dst, zeros128)
    y, s = _tc_mid(agg, y, s, dinv, conv_b0, bn_g0, bn_b0, ln_g0, ln_b0,
                   conv_W1, lin_W1, lin_b1)
    agg = _sc_agg(y, src, dst, zeros128)
    y, s = _tc_mid(agg, y, s, dinv, conv_b1, bn_g1, bn_b1, ln_g1, ln_b1,
                   conv_W2, lin_W2, lin_b2)
    agg = _sc_agg(y, src, dst, zeros128)
    out = _tc_final(agg, y, s, dinv, conv_b2, bn_g2, bn_b2, ln_g2, ln_b2,
                    h0, pW, pb)
    return out[:, :OUT]


# speedup vs baseline: 8.3792x; 1.0769x over previous
import functools

import jax
import jax.numpy as jnp
from jax import lax
from jax.experimental import pallas as pl
from jax.experimental.pallas import tpu as pltpu
from jax.experimental.pallas import tpu_sc as plsc

N = 10000
E = 320000
HID = 128
OUT = 7

NW = 32
CHUNK = 128
EPT = 10240
NCH = EPT // CHUNK
NACC = 10112
RPT = NACC // 16


def _sc_degree_body(dst_hbm, zeros_hbm, ones_hbm, out_hbm, dst_v, ones_v, slab):
    cid = lax.axis_index("c")
    sid = lax.axis_index("s")
    wid = sid * 2 + cid
    pltpu.sync_copy(dst_hbm.at[wid], dst_v)
    pltpu.sync_copy(ones_hbm, ones_v)
    pltpu.sync_copy(zeros_hbm.at[pl.ds(sid * RPT, RPT)],
                    slab.at[pl.ds(sid * RPT, RPT)])
    plsc.subcore_barrier()

    def body(j, _):
        pltpu.sync_copy(ones_v, slab.at[dst_v.at[j]], add=True)
        return ()

    lax.fori_loop(0, NCH, body, ())
    plsc.subcore_barrier()
    pltpu.sync_copy(slab.at[pl.ds(sid * RPT, RPT)],
                    out_hbm.at[cid, pl.ds(sid * RPT, RPT)])


NBUF = 2
NROUND = NCH // NBUF


def _sc_agg_body(y_hbm, src_hbm, dst_hbm, zeros_hbm, out_hbm,
                 src_v, dwin, buf0, buf1,
                 g0, g1, s0, s1, d0, d1, acc):
    bufs = (buf0, buf1)
    gsems = (g0, g1)
    ssems = (s0, s1)
    dsems = (d0, d1)
    cid = lax.axis_index("c")
    sid = lax.axis_index("s")
    wid = sid * 2 + cid
    pltpu.sync_copy(src_hbm.at[wid], src_v)
    pltpu.sync_copy(zeros_hbm.at[pl.ds(sid * RPT, RPT)],
                    acc.at[pl.ds(sid * RPT, RPT)])
    plsc.subcore_barrier()

    def gather(j, b):
        jc = jnp.minimum(j, NCH - 1)
        return pltpu.make_async_copy(y_hbm.at[src_v.at[jc]], bufs[b], gsems[b])

    def dload(r, p):
        base = jnp.minimum(r * NBUF, NCH - NBUF)
        return pltpu.make_async_copy(dst_hbm.at[wid, pl.ds(base, NBUF)],
                                     dwin.at[p], dsems[p])

    dload(0, 0).start()
    dload(1, 1).start()
    gather(0, 0).start()
    gather(1, 1).start()

    def half(r, p):
        jb = r * NBUF
        dload(r, p).wait()
        descs = []
        for b in range(NBUF):
            gather(jb + b, b).wait()
            descs.append(pltpu.async_copy(bufs[b], acc.at[dwin.at[p, b]],
                                          ssems[b], add=True))
        for b in range(NBUF):
            descs[b].wait()
            gather(jb + NBUF + b, b).start()
        dload(r + 2, p).start()

    def body(t, _):
        half(2 * t, 0)
        half(2 * t + 1, 1)
        return ()

    lax.fori_loop(0, NROUND // 2, body, ())
    for b in range(NBUF):
        gather(NCH - 1, b).wait()
    for p in range(2):
        dload(0, p).wait()
    plsc.subcore_barrier()
    pltpu.sync_copy(acc.at[pl.ds(sid * RPT, RPT)],
                    out_hbm.at[cid, pl.ds(sid * RPT, RPT)])


@functools.cache
def _sc_kernels():
    mesh = plsc.VectorSubcoreMesh(core_axis_name="c", subcore_axis_name="s")
    sc_degree = pl.kernel(
        _sc_degree_body,
        out_type=jax.ShapeDtypeStruct((2, NACC, HID), jnp.float32),
        mesh=mesh,
        scratch_types=[
            pltpu.VMEM((NCH, CHUNK), jnp.int32),
            pltpu.VMEM((CHUNK, HID), jnp.float32),
            pltpu.VMEM_SHARED((NACC, HID), jnp.float32),
        ],
    )
    sc_agg = pl.kernel(
        _sc_agg_body,
        out_type=jax.ShapeDtypeStruct((2, NACC, HID), jnp.float32),
        mesh=mesh,
        scratch_types=[
            pltpu.VMEM((NCH, CHUNK), jnp.int32),
            pltpu.VMEM((2, NBUF, CHUNK), jnp.int32),
        ] + [pltpu.VMEM((CHUNK, HID), jnp.float32)] * NBUF
          + [pltpu.SemaphoreType.DMA] * (2 * NBUF + 2)
          + [pltpu.VMEM_SHARED((NACC, HID), jnp.float32)],
    )
    return sc_degree, sc_agg


def _tc0_body(x_ref, slab_ref, liW_ref, lib_ref, cW_ref, lW_ref, lb_ref,
              h0_ref, y_ref, s_ref, dinv_ref):
    x = x_ref[...]
    deg = 1.0 + slab_ref[0, :N, 0:1] + slab_ref[1, :N, 0:1]
    dinv = lax.rsqrt(deg)
    dinv_ref[...] = dinv
    h0_ref[...] = jnp.maximum(
        jnp.dot(x, liW_ref[...], preferred_element_type=jnp.float32)
        + lib_ref[...], 0.0)
    y_ref[...] = dinv * jnp.dot(x, cW_ref[...],
                                preferred_element_type=jnp.float32)
    s_ref[...] = jnp.dot(x, lW_ref[...],
                         preferred_element_type=jnp.float32) + lb_ref[...]


def _norm_relu(agg_ref, y_ref, s_ref, dinv_ref, cb_ref,
               bng_ref, bnb_ref, lng_ref, lnb_ref):
    dinv = dinv_ref[...]
    agg = agg_ref[0, :N, :] + agg_ref[1, :N, :]
    c = dinv * (agg + y_ref[...]) + cb_ref[...]
    t = c + s_ref[...]
    m = jnp.mean(t, axis=0, keepdims=True)
    d = t - m
    v = jnp.mean(d * d, axis=0, keepdims=True)
    tb = bng_ref[...] * d * lax.rsqrt(v + 1e-5) + bnb_ref[...]
    m2 = jnp.mean(tb, axis=1, keepdims=True)
    d2 = tb - m2
    v2 = jnp.mean(d2 * d2, axis=1, keepdims=True)
    tl = lng_ref[...] * d2 * lax.rsqrt(v2 + 1e-5) + lnb_ref[...]
    return jnp.maximum(tl, 0.0)


def _tc_mid_body(agg_ref, y_ref, s_ref, dinv_ref, cb_ref,
                 bng_ref, bnb_ref, lng_ref, lnb_ref,
                 cW2_ref, lW2_ref, lb2_ref,
                 y2_ref, s2_ref):
    h = _norm_relu(agg_ref, y_ref, s_ref, dinv_ref, cb_ref,
                   bng_ref, bnb_ref, lng_ref, lnb_ref)
    dinv = dinv_ref[...]
    y2_ref[...] = dinv * jnp.dot(h, cW2_ref[...],
                                 preferred_element_type=jnp.float32)
    s2_ref[...] = jnp.dot(h, lW2_ref[...],
                          preferred_element_type=jnp.float32) + lb2_ref[...]


def _tc_final_body(agg_ref, y_ref, s_ref, dinv_ref, cb_ref,
                   bng_ref, bnb_ref, lng_ref, lnb_ref,
                   h0_ref, pW_ref, pb_ref, out_ref):
    h = _norm_relu(agg_ref, y_ref, s_ref, dinv_ref, cb_ref,
                   bng_ref, bnb_ref, lng_ref, lnb_ref)
    out_ref[...] = jnp.dot(h + h0_ref[...], pW_ref[...],
                           preferred_element_type=jnp.float32) + pb_ref[...]


_f32 = lambda *s: jax.ShapeDtypeStruct(s, jnp.float32)

_tc0 = pl.pallas_call(
    _tc0_body,
    out_shape=(_f32(N, HID), _f32(N, HID), _f32(N, HID), _f32(N, 1)),
)

_tc_mid = pl.pallas_call(
    _tc_mid_body,
    out_shape=(_f32(N, HID), _f32(N, HID)),
)

_tc_final = pl.pallas_call(
    _tc_final_body,
    out_shape=_f32(N, 8),
)


def kernel(x, edge_index,
           lin_in_W, lin_in_b,
           conv_W0, conv_b0, conv_W1, conv_b1, conv_W2, conv_b2,
           lin_W0, lin_b0, lin_W1, lin_b1, lin_W2, lin_b2,
           ln_g0, ln_b0, ln_g1, ln_b1, ln_g2, ln_b2,
           bn_g0, bn_b0, bn_g1, bn_b1, bn_g2, bn_b2,
           pred_W, pred_b):
    pad = NW * EPT - E
    src = jnp.concatenate(
        [edge_index[0], jnp.zeros((pad,), jnp.int32)]).reshape(NW, NCH, CHUNK)
    dst = jnp.concatenate(
        [edge_index[1], jnp.full((pad,), N, jnp.int32)]).reshape(NW, NCH, CHUNK)
    zeros128 = jnp.zeros((NACC, HID), jnp.float32)
    ones128 = jnp.ones((CHUNK, HID), jnp.float32)
    pW = jnp.zeros((HID, 8), jnp.float32).at[:, :OUT].set(pred_W)
    pb = jnp.zeros((8,), jnp.float32).at[:OUT].set(pred_b)

    _sc_degree, _sc_agg = _sc_kernels()
    slab = _sc_degree(dst, zeros128, ones128)
    h0, y, s, dinv = _tc0(x, slab, lin_in_W, lin_in_b, conv_W0, lin_W0, lin_b0)

    agg = _sc_agg(y, src, dst, zeros128)
    y, s = _tc_mid(agg, y, s, dinv, conv_b0, bn_g0, bn_b0, ln_g0, ln_b0,
                   conv_W1, lin_W1, lin_b1)
    agg = _sc_agg(y, src, dst, zeros128)
    y, s = _tc_mid(agg, y, s, dinv, conv_b1, bn_g1, bn_b1, ln_g1, ln_b1,
                   conv_W2, lin_W2, lin_b2)
    agg = _sc_agg(y, src, dst, zeros128)
    out = _tc_final(agg, y, s, dinv, conv_b2, bn_g2, bn_b2, ln_g2, ln_b2,
                    h0, pW, pb)
    return out[:, :OUT]


# speedup vs baseline: 9.0361x; 1.0784x over previous
import functools

import jax
import jax.numpy as jnp
from jax import lax
from jax.experimental import pallas as pl
from jax.experimental.pallas import tpu as pltpu
from jax.experimental.pallas import tpu_sc as plsc

N = 10000
E = 320000
HID = 128
OUT = 7

NW = 32
CHUNK = 128
NPAIR = 16
NCHP = 160
EPT = NCHP * CHUNK // 2
NCH = 80
NACC = 10112
RPT = NACC // 16
C0 = 120
C1 = NCHP - C0


def _sc_degree_body(dst_hbm, zeros_hbm, ones_hbm, out_hbm, dst_v, ones_v, slab):
    cid = lax.axis_index("c")
    sid = lax.axis_index("s")
    pltpu.sync_copy(dst_hbm.at[sid, pl.ds(cid * NCH, NCH)], dst_v)
    pltpu.sync_copy(ones_hbm, ones_v)
    pltpu.sync_copy(zeros_hbm.at[pl.ds(sid * RPT, RPT)],
                    slab.at[pl.ds(sid * RPT, RPT)])
    plsc.subcore_barrier()

    def body(j, _):
        pltpu.sync_copy(ones_v, slab.at[dst_v.at[j]], add=True)
        return ()

    lax.fori_loop(0, NCH, body, ())
    plsc.subcore_barrier()
    pltpu.sync_copy(slab.at[pl.ds(sid * RPT, RPT)],
                    out_hbm.at[cid, pl.ds(sid * RPT, RPT)])


NBUF = 2
NROUND = NCH // NBUF


def _sc_agg_body(y_hbm, src_hbm, dst_hbm, zeros_hbm, out_hbm,
                 src_v, dwin, buf0, buf1,
                 g0, g1, s0, s1, d0, d1, acc):
    bufs = (buf0, buf1)
    gsems = (g0, g1)
    ssems = (s0, s1)
    dsems = (d0, d1)
    cid = lax.axis_index("c")
    sid = lax.axis_index("s")
    pltpu.sync_copy(zeros_hbm.at[pl.ds(sid * RPT, RPT)],
                    acc.at[pl.ds(sid * RPT, RPT)])
    plsc.subcore_barrier()

    def run_range(start, count):
        pltpu.sync_copy(src_hbm.at[sid, pl.ds(start, count)],
                        src_v.at[pl.ds(0, count)])

        def gather(j, b):
            jc = jnp.minimum(j, count - 1)
            return pltpu.make_async_copy(y_hbm.at[src_v.at[jc]],
                                         bufs[b], gsems[b])

        def dload(r, p):
            base = start + jnp.minimum(r * NBUF, count - NBUF)
            return pltpu.make_async_copy(dst_hbm.at[sid, pl.ds(base, NBUF)],
                                         dwin.at[p], dsems[p])

        dload(0, 0).start()
        dload(1, 1).start()
        gather(0, 0).start()
        gather(1, 1).start()

        def half(r, p):
            jb = r * NBUF
            dload(r, p).wait()
            descs = []
            for b in range(NBUF):
                gather(jb + b, b).wait()
                descs.append(pltpu.async_copy(bufs[b], acc.at[dwin.at[p, b]],
                                              ssems[b], add=True))
            for b in range(NBUF):
                descs[b].wait()
                gather(jb + NBUF + b, b).start()
            dload(r + 2, p).start()

        def body(t, _):
            half(2 * t, 0)
            half(2 * t + 1, 1)
            return ()

        lax.fori_loop(0, count // (2 * NBUF), body, ())
        for b in range(NBUF):
            gather(count - 1, b).wait()
        for p in range(2):
            dload(0, p).wait()

    @pl.when(cid == 0)
    def _():
        run_range(0, C0)

    @pl.when(cid == 1)
    def _():
        run_range(C0, C1)

    plsc.subcore_barrier()
    pltpu.sync_copy(acc.at[pl.ds(sid * RPT, RPT)],
                    out_hbm.at[cid, pl.ds(sid * RPT, RPT)])


@functools.cache
def _sc_kernels():
    mesh = plsc.VectorSubcoreMesh(core_axis_name="c", subcore_axis_name="s")
    sc_degree = pl.kernel(
        _sc_degree_body,
        out_type=jax.ShapeDtypeStruct((2, NACC, HID), jnp.float32),
        mesh=mesh,
        scratch_types=[
            pltpu.VMEM((NCH, CHUNK), jnp.int32),
            pltpu.VMEM((CHUNK, HID), jnp.float32),
            pltpu.VMEM_SHARED((NACC, HID), jnp.float32),
        ],
    )
    sc_agg = pl.kernel(
        _sc_agg_body,
        out_type=jax.ShapeDtypeStruct((2, NACC, HID), jnp.float32),
        mesh=mesh,
        scratch_types=[
            pltpu.VMEM((C0, CHUNK), jnp.int32),
            pltpu.VMEM((2, NBUF, CHUNK), jnp.int32),
        ] + [pltpu.VMEM((CHUNK, HID), jnp.float32)] * NBUF
          + [pltpu.SemaphoreType.DMA] * (2 * NBUF + 2)
          + [pltpu.VMEM_SHARED((NACC, HID), jnp.float32)],
    )
    return sc_degree, sc_agg


def _tc0_body(x_ref, slab_ref, liW_ref, lib_ref, cW_ref, lW_ref, lb_ref,
              h0_ref, y_ref, s_ref, dinv_ref):
    x = x_ref[...]
    deg = 1.0 + slab_ref[0, :N, 0:1] + slab_ref[1, :N, 0:1]
    dinv = lax.rsqrt(deg)
    dinv_ref[...] = dinv
    h0_ref[...] = jnp.maximum(
        jnp.dot(x, liW_ref[...], preferred_element_type=jnp.float32)
        + lib_ref[...], 0.0)
    y_ref[...] = dinv * jnp.dot(x, cW_ref[...],
                                preferred_element_type=jnp.float32)
    s_ref[...] = jnp.dot(x, lW_ref[...],
                         preferred_element_type=jnp.float32) + lb_ref[...]


def _norm_relu(agg_ref, y_ref, s_ref, dinv_ref, cb_ref,
               bng_ref, bnb_ref, lng_ref, lnb_ref):
    dinv = dinv_ref[...]
    agg = agg_ref[0, :N, :] + agg_ref[1, :N, :]
    c = dinv * (agg + y_ref[...]) + cb_ref[...]
    t = c + s_ref[...]
    m = jnp.mean(t, axis=0, keepdims=True)
    d = t - m
    v = jnp.mean(d * d, axis=0, keepdims=True)
    tb = bng_ref[...] * d * lax.rsqrt(v + 1e-5) + bnb_ref[...]
    m2 = jnp.mean(tb, axis=1, keepdims=True)
    d2 = tb - m2
    v2 = jnp.mean(d2 * d2, axis=1, keepdims=True)
    tl = lng_ref[...] * d2 * lax.rsqrt(v2 + 1e-5) + lnb_ref[...]
    return jnp.maximum(tl, 0.0)


def _tc_mid_body(agg_ref, y_ref, s_ref, dinv_ref, cb_ref,
                 bng_ref, bnb_ref, lng_ref, lnb_ref,
                 cW2_ref, lW2_ref, lb2_ref,
                 y2_ref, s2_ref):
    h = _norm_relu(agg_ref, y_ref, s_ref, dinv_ref, cb_ref,
                   bng_ref, bnb_ref, lng_ref, lnb_ref)
    dinv = dinv_ref[...]
    y2_ref[...] = dinv * jnp.dot(h, cW2_ref[...],
                                 preferred_element_type=jnp.float32)
    s2_ref[...] = jnp.dot(h, lW2_ref[...],
                          preferred_element_type=jnp.float32) + lb2_ref[...]


def _tc_final_body(agg_ref, y_ref, s_ref, dinv_ref, cb_ref,
                   bng_ref, bnb_ref, lng_ref, lnb_ref,
                   h0_ref, pW_ref, pb_ref, out_ref):
    h = _norm_relu(agg_ref, y_ref, s_ref, dinv_ref, cb_ref,
                   bng_ref, bnb_ref, lng_ref, lnb_ref)
    out_ref[...] = jnp.dot(h + h0_ref[...], pW_ref[...],
                           preferred_element_type=jnp.float32) + pb_ref[...]


_f32 = lambda *s: jax.ShapeDtypeStruct(s, jnp.float32)

_tc0 = pl.pallas_call(
    _tc0_body,
    out_shape=(_f32(N, HID), _f32(N, HID), _f32(N, HID), _f32(N, 1)),
)

_tc_mid = pl.pallas_call(
    _tc_mid_body,
    out_shape=(_f32(N, HID), _f32(N, HID)),
)

_tc_final = pl.pallas_call(
    _tc_final_body,
    out_shape=_f32(N, 8),
)


def kernel(x, edge_index,
           lin_in_W, lin_in_b,
           conv_W0, conv_b0, conv_W1, conv_b1, conv_W2, conv_b2,
           lin_W0, lin_b0, lin_W1, lin_b1, lin_W2, lin_b2,
           ln_g0, ln_b0, ln_g1, ln_b1, ln_g2, ln_b2,
           bn_g0, bn_b0, bn_g1, bn_b1, bn_g2, bn_b2,
           pred_W, pred_b):
    pad = NPAIR * NCHP * CHUNK - E
    src = jnp.concatenate(
        [edge_index[0], jnp.zeros((pad,), jnp.int32)]).reshape(NPAIR, NCHP, CHUNK)
    dst = jnp.concatenate(
        [edge_index[1], jnp.full((pad,), N, jnp.int32)]).reshape(NPAIR, NCHP, CHUNK)
    zeros128 = jnp.zeros((NACC, HID), jnp.float32)
    ones128 = jnp.ones((CHUNK, HID), jnp.float32)
    pW = jnp.zeros((HID, 8), jnp.float32).at[:, :OUT].set(pred_W)
    pb = jnp.zeros((8,), jnp.float32).at[:OUT].set(pred_b)

    _sc_degree, _sc_agg = _sc_kernels()
    slab = _sc_degree(dst, zeros128, ones128)
    h0, y, s, dinv = _tc0(x, slab, lin_in_W, lin_in_b, conv_W0, lin_W0, lin_b0)

    agg = _sc_agg(y, src, dst, zeros128)
    y, s = _tc_mid(agg, y, s, dinv, conv_b0, bn_g0, bn_b0, ln_g0, ln_b0,
                   conv_W1, lin_W1, lin_b1)
    agg = _sc_agg(y, src, dst, zeros128)
    y, s = _tc_mid(agg, y, s, dinv, conv_b1, bn_g1, bn_b1, ln_g1, ln_b1,
                   conv_W2, lin_W2, lin_b2)
    agg = _sc_agg(y, src, dst, zeros128)
    out = _tc_final(agg, y, s, dinv, conv_b2, bn_g2, bn_b2, ln_g2, ln_b2,
                    h0, pW, pb)
    return out[:, :OUT]


# speedup vs baseline: 9.0426x; 1.0007x over previous
import functools

import jax
import jax.numpy as jnp
from jax import lax
from jax.experimental import pallas as pl
from jax.experimental.pallas import tpu as pltpu
from jax.experimental.pallas import tpu_sc as plsc

N = 10000
E = 320000
HID = 128
OUT = 7

CHUNK = 128
NPAIR = 16
NCHP = 160
NCH = NCHP // 2
NACC = 10112
RPT = NACC // 16
C0 = 120
C1 = NCHP - C0


def _sc_degree_body(dst_hbm, zeros_hbm, ones_hbm, out_hbm, dst_v, ones_v, slab):
    cid = lax.axis_index("c")
    sid = lax.axis_index("s")
    pltpu.sync_copy(dst_hbm.at[sid, pl.ds(cid * NCH, NCH)], dst_v)
    pltpu.sync_copy(ones_hbm, ones_v)
    pltpu.sync_copy(zeros_hbm.at[pl.ds(sid * RPT, RPT)],
                    slab.at[pl.ds(sid * RPT, RPT)])
    plsc.subcore_barrier()

    def body(j, _):
        pltpu.sync_copy(ones_v, slab.at[dst_v.at[j]], add=True)
        return ()

    lax.fori_loop(0, NCH, body, ())
    plsc.subcore_barrier()
    pltpu.sync_copy(slab.at[pl.ds(sid * RPT, RPT)],
                    out_hbm.at[cid, pl.ds(sid * RPT, RPT)])


NBUF = 2


def _sc_agg_body(y_hbm, src_hbm, dst_hbm, zeros_hbm, out_hbm,
                 src_v, dwin, buf0, buf1,
                 g0, g1, s0, s1, d0, d1, acc):
    bufs = (buf0, buf1)
    gsems = (g0, g1)
    ssems = (s0, s1)
    dsems = (d0, d1)
    cid = lax.axis_index("c")
    sid = lax.axis_index("s")
    pltpu.sync_copy(zeros_hbm.at[pl.ds(sid * RPT, RPT)],
                    acc.at[pl.ds(sid * RPT, RPT)])
    plsc.subcore_barrier()

    def run_range(start, count):
        pltpu.sync_copy(src_hbm.at[sid, pl.ds(start, count)],
                        src_v.at[pl.ds(0, count)])

        def gather(j, b):
            jc = jnp.minimum(j, count - 1)
            return pltpu.make_async_copy(y_hbm.at[src_v.at[jc]],
                                         bufs[b], gsems[b])

        def dload(r, p):
            base = start + jnp.minimum(r * NBUF, count - NBUF)
            return pltpu.make_async_copy(dst_hbm.at[sid, pl.ds(base, NBUF)],
                                         dwin.at[p], dsems[p])

        dload(0, 0).start()
        dload(1, 1).start()
        gather(0, 0).start()
        gather(1, 1).start()

        def half(r, p):
            jb = r * NBUF
            dload(r, p).wait()
            descs = []
            for b in range(NBUF):
                gather(jb + b, b).wait()
                descs.append(pltpu.async_copy(bufs[b], acc.at[dwin.at[p, b]],
                                              ssems[b], add=True))
            for b in range(NBUF):
                descs[b].wait()
                gather(jb + NBUF + b, b).start()
            dload(r + 2, p).start()

        def body(t, _):
            half(2 * t, 0)
            half(2 * t + 1, 1)
            return ()

        lax.fori_loop(0, count // (2 * NBUF), body, ())
        for b in range(NBUF):
            gather(count - 1, b).wait()
        for p in range(2):
            dload(0, p).wait()

    @pl.when(cid == 0)
    def _():
        run_range(0, C0)

    @pl.when(cid == 1)
    def _():
        run_range(C0, C1)

    plsc.subcore_barrier()
    pltpu.sync_copy(acc.at[pl.ds(sid * RPT, RPT)],
                    out_hbm.at[cid, pl.ds(sid * RPT, RPT)])


@functools.cache
def _sc_kernels():
    mesh = plsc.VectorSubcoreMesh(core_axis_name="c", subcore_axis_name="s")
    sc_degree = pl.kernel(
        _sc_degree_body,
        out_type=jax.ShapeDtypeStruct((2, NACC, HID), jnp.float32),
        mesh=mesh,
        scratch_types=[
            pltpu.VMEM((NCH, CHUNK), jnp.int32),
            pltpu.VMEM((CHUNK, HID), jnp.float32),
            pltpu.VMEM_SHARED((NACC, HID), jnp.float32),
        ],
    )
    sc_agg = pl.kernel(
        _sc_agg_body,
        out_type=jax.ShapeDtypeStruct((2, NACC, HID), jnp.float32),
        mesh=mesh,
        scratch_types=[
            pltpu.VMEM((C0, CHUNK), jnp.int32),
            pltpu.VMEM((2, NBUF, CHUNK), jnp.int32),
        ] + [pltpu.VMEM((CHUNK, HID), jnp.float32)] * NBUF
          + [pltpu.SemaphoreType.DMA] * (2 * NBUF + 2)
          + [pltpu.VMEM_SHARED((NACC, HID), jnp.float32)],
    )
    return sc_degree, sc_agg


def _tc0_body(x_ref, slab_ref, liW_ref, lib_ref, cW_ref, lW_ref, lb_ref,
              h0_ref, y_ref, s_ref, dinv_ref):
    x = x_ref[...]
    deg = 1.0 + slab_ref[0, :N, 0:1] + slab_ref[1, :N, 0:1]
    dinv = lax.rsqrt(deg)
    dinv_ref[...] = dinv
    h0_ref[...] = jnp.maximum(
        jnp.dot(x, liW_ref[...], preferred_element_type=jnp.float32)
        + lib_ref[...], 0.0)
    y_ref[...] = dinv * jnp.dot(x, cW_ref[...],
                                preferred_element_type=jnp.float32)
    s_ref[...] = jnp.dot(x, lW_ref[...],
                         preferred_element_type=jnp.float32) + lb_ref[...]


def _norm_relu(agg_ref, y_ref, s_ref, dinv_ref, cb_ref,
               bng_ref, bnb_ref, lng_ref, lnb_ref):
    dinv = dinv_ref[...]
    agg = agg_ref[0, :N, :] + agg_ref[1, :N, :]
    c = dinv * (agg + y_ref[...]) + cb_ref[...]
    t = c + s_ref[...]
    m = jnp.mean(t, axis=0, keepdims=True)
    d = t - m
    v = jnp.mean(d * d, axis=0, keepdims=True)
    tb = bng_ref[...] * d * lax.rsqrt(v + 1e-5) + bnb_ref[...]
    m2 = jnp.mean(tb, axis=1, keepdims=True)
    d2 = tb - m2
    v2 = jnp.mean(d2 * d2, axis=1, keepdims=True)
    tl = lng_ref[...] * d2 * lax.rsqrt(v2 + 1e-5) + lnb_ref[...]
    return jnp.maximum(tl, 0.0)


def _tc_mid_body(agg_ref, y_ref, s_ref, dinv_ref, cb_ref,
                 bng_ref, bnb_ref, lng_ref, lnb_ref,
                 cW2_ref, lW2_ref, lb2_ref,
                 y2_ref, s2_ref):
    h = _norm_relu(agg_ref, y_ref, s_ref, dinv_ref, cb_ref,
                   bng_ref, bnb_ref, lng_ref, lnb_ref)
    dinv = dinv_ref[...]
    y2_ref[...] = dinv * jnp.dot(h, cW2_ref[...],
                                 preferred_element_type=jnp.float32)
    s2_ref[...] = jnp.dot(h, lW2_ref[...],
                          preferred_element_type=jnp.float32) + lb2_ref[...]


def _tc_final_body(agg_ref, y_ref, s_ref, dinv_ref, cb_ref,
                   bng_ref, bnb_ref, lng_ref, lnb_ref,
                   h0_ref, pW_ref, pb_ref, out_ref):
    h = _norm_relu(agg_ref, y_ref, s_ref, dinv_ref, cb_ref,
                   bng_ref, bnb_ref, lng_ref, lnb_ref)
    out_ref[...] = jnp.dot(h + h0_ref[...], pW_ref[...],
                           preferred_element_type=jnp.float32) + pb_ref[...]


_f32 = lambda *s: jax.ShapeDtypeStruct(s, jnp.float32)

_tc0 = pl.pallas_call(
    _tc0_body,
    out_shape=(_f32(N, HID), _f32(N, HID), _f32(N, HID), _f32(N, 1)),
)

_tc_mid = pl.pallas_call(
    _tc_mid_body,
    out_shape=(_f32(N, HID), _f32(N, HID)),
)

_tc_final = pl.pallas_call(
    _tc_final_body,
    out_shape=_f32(N, 8),
)


def kernel(x, edge_index,
           lin_in_W, lin_in_b,
           conv_W0, conv_b0, conv_W1, conv_b1, conv_W2, conv_b2,
           lin_W0, lin_b0, lin_W1, lin_b1, lin_W2, lin_b2,
           ln_g0, ln_b0, ln_g1, ln_b1, ln_g2, ln_b2,
           bn_g0, bn_b0, bn_g1, bn_b1, bn_g2, bn_b2,
           pred_W, pred_b):
    pad = NPAIR * NCHP * CHUNK - E
    src = jnp.concatenate(
        [edge_index[0], jnp.zeros((pad,), jnp.int32)]).reshape(NPAIR, NCHP, CHUNK)
    dst = jnp.concatenate(
        [edge_index[1], jnp.full((pad,), N, jnp.int32)]).reshape(NPAIR, NCHP, CHUNK)
    zeros128 = jnp.zeros((NACC, HID), jnp.float32)
    ones128 = jnp.ones((CHUNK, HID), jnp.float32)
    pW = jnp.zeros((HID, 8), jnp.float32).at[:, :OUT].set(pred_W)
    pb = jnp.zeros((8,), jnp.float32).at[:OUT].set(pred_b)

    _sc_degree, _sc_agg = _sc_kernels()
    slab = _sc_degree(dst, zeros128, ones128)
    h0, y, s, dinv = _tc0(x, slab, lin_in_W, lin_in_b, conv_W0, lin_W0, lin_b0)

    agg = _sc_agg(y, src, dst, zeros128)
    y, s = _tc_mid(agg, y, s, dinv, conv_b0, bn_g0, bn_b0, ln_g0, ln_b0,
                   conv_W1, lin_W1, lin_b1)
    agg = _sc_agg(y, src, dst, zeros128)
    y, s = _tc_mid(agg, y, s, dinv, conv_b1, bn_g1, bn_b1, ln_g1, ln_b1,
                   conv_W2, lin_W2, lin_b2)
    agg = _sc_agg(y, src, dst, zeros128)
    out = _tc_final(agg, y, s, dinv, conv_b2, bn_g2, bn_b2, ln_g2, ln_b2,
                    h0, pW, pb)
    return out[:, :OUT]
